# baseline (speedup 1.0000x reference)
import functools

import jax
import jax.numpy as jnp
from jax import lax
from jax.experimental import pallas as pl
from jax.experimental.pallas import tpu as pltpu
from jax.experimental.pallas import tpu_sc as plsc

_NC = 2
_NS = 16
_NW = _NC * _NS


def _pick_chunk(per_worker: int) -> int:
    for k in (8000, 6400, 5000, 4000, 3200, 2000, 1600, 1000, 800, 500, 400,
              250, 200, 125, 100, 64, 40, 32, 16, 8):
        if per_worker % k == 0 and k % 8 == 0:
            return k
    return 8


def _sc_deg(row, attr, np_pad):
    e = row.shape[0]
    ce = e // _NW
    k = _pick_chunk(ce)
    ntile = np_pad // _NS
    mesh = plsc.VectorSubcoreMesh(core_axis_name="c", subcore_axis_name="s")

    @functools.partial(
        pl.kernel,
        out_type=jax.ShapeDtypeStruct((_NC, np_pad), jnp.float32),
        mesh=mesh,
        scratch_types=[
            pltpu.VMEM_SHARED((np_pad,), jnp.float32),
            pltpu.VMEM((k,), jnp.int32),
            pltpu.VMEM((k,), jnp.float32),
            pltpu.VMEM((ntile,), jnp.float32),
        ],
    )
    def deg_kernel(row_h, attr_h, out_h, deg_sh, idx_v, val_v, zero_v):
        c = lax.axis_index("c")
        s = lax.axis_index("s")
        wid = c * _NS + s

        def zero_body(i, _):
            zero_v[pl.ds(i * 16, 16)] = jnp.zeros((16,), jnp.float32)
            return _

        lax.fori_loop(0, ntile // 16, zero_body, None, unroll=4)
        pltpu.sync_copy(zero_v, deg_sh.at[pl.ds(s * ntile, ntile)])
        plsc.subcore_barrier()

        def chunk_body(i, _):
            base = wid * ce + i * k
            pltpu.sync_copy(row_h.at[pl.ds(base, k)], idx_v)
            pltpu.sync_copy(attr_h.at[pl.ds(base, k)], val_v)
            pltpu.sync_copy(val_v, deg_sh.at[idx_v], add=True)
            return _

        lax.fori_loop(0, ce // k, chunk_body, None)
        plsc.subcore_barrier()
        pltpu.sync_copy(deg_sh.at[pl.ds(s * ntile, ntile)],
                        out_h.at[c, pl.ds(s * ntile, ntile)])

    return deg_kernel(row, attr)


def _sc_prop(row, col, attr, y0, y1, np_pad):
    e = row.shape[0]
    ce = e // _NW
    k = 5000
    nloop = ce // (2 * k)
    ntile = np_pad // _NS
    mesh = plsc.VectorSubcoreMesh(core_axis_name="c", subcore_axis_name="s")

    @functools.partial(
        pl.kernel,
        out_type=jax.ShapeDtypeStruct((_NC, 2, np_pad), jnp.float32),
        mesh=mesh,
        scratch_types=[
            pltpu.VMEM_SHARED((np_pad,), jnp.float32),
            pltpu.VMEM_SHARED((np_pad,), jnp.float32),
            pltpu.VMEM_SHARED((np_pad,), jnp.float32),
            pltpu.VMEM_SHARED((np_pad,), jnp.float32),
            pltpu.VMEM((k,), jnp.int32),
            pltpu.VMEM((k,), jnp.int32),
            pltpu.VMEM((k,), jnp.int32),
            pltpu.VMEM((k,), jnp.int32),
            pltpu.VMEM((k,), jnp.float32),
            pltpu.VMEM((k,), jnp.float32),
            pltpu.VMEM((k,), jnp.float32),
            pltpu.VMEM((k,), jnp.float32),
            pltpu.VMEM((k,), jnp.float32),
            pltpu.VMEM((k,), jnp.float32),
            pltpu.VMEM((ntile,), jnp.float32),
            pltpu.SemaphoreType.DMA,
            pltpu.SemaphoreType.DMA,
            pltpu.SemaphoreType.DMA,
            pltpu.SemaphoreType.DMA,
            pltpu.SemaphoreType.DMA,
            pltpu.SemaphoreType.DMA,
        ],
    )
    def prop_kernel(row_h, col_h, attr_h, y0_h, y1_h, out_h,
                    y0_sh, y1_sh, z0_sh, z1_sh,
                    row_vA, row_vB, col_vA, col_vB, attr_vA, attr_vB,
                    g0_vA, g0_vB, g1_vA, g1_vB, zero_v,
                    st_semA, st_semB, g_semA, g_semB, sc_semA, sc_semB):
        c = lax.axis_index("c")
        s = lax.axis_index("s")
        wid = c * _NS + s
        tsl = pl.ds(s * ntile, ntile)
        row_v = (row_vA, row_vB)
        col_v = (col_vA, col_vB)
        attr_v = (attr_vA, attr_vB)
        g0_v = (g0_vA, g0_vB)
        g1_v = (g1_vA, g1_vB)
        st_sem = (st_semA, st_semB)
        g_sem = (g_semA, g_semB)
        sc_sem = (sc_semA, sc_semB)

        pltpu.sync_copy(y0_h.at[tsl], y0_sh.at[tsl])
        pltpu.sync_copy(y1_h.at[tsl], y1_sh.at[tsl])

        def zero_body(i, _):
            zero_v[pl.ds(i * 16, 16)] = jnp.zeros((16,), jnp.float32)
            return _

        lax.fori_loop(0, ntile // 16, zero_body, None, unroll=4)
        pltpu.sync_copy(zero_v, z0_sh.at[tsl])
        pltpu.sync_copy(zero_v, z1_sh.at[tsl])
        plsc.subcore_barrier()

        def fire_streams(b, base):
            pltpu.async_copy(row_h.at[pl.ds(base, k)], row_v[b], st_sem[b])
            pltpu.async_copy(col_h.at[pl.ds(base, k)], col_v[b], st_sem[b])
            pltpu.async_copy(attr_h.at[pl.ds(base, k)], attr_v[b], st_sem[b])

        def drain_streams(b):
            pltpu.make_async_copy(row_h.at[pl.ds(0, k)], row_v[b],
                                  st_sem[b]).wait()
            pltpu.make_async_copy(col_h.at[pl.ds(0, k)], col_v[b],
                                  st_sem[b]).wait()
            pltpu.make_async_copy(attr_h.at[pl.ds(0, k)], attr_v[b],
                                  st_sem[b]).wait()

        def drain_scatters(b):
            pltpu.make_async_copy(attr_h.at[pl.ds(0, k)], g0_v[b],
                                  sc_sem[b]).wait()
            pltpu.make_async_copy(attr_h.at[pl.ds(0, k)], g1_v[b],
                                  sc_sem[b]).wait()

        def fire_gathers(b):
            pltpu.async_copy(y0_sh.at[row_v[b]], g0_v[b], g_sem[b])
            pltpu.async_copy(y1_sh.at[row_v[b]], g1_v[b], g_sem[b])
            pltpu.make_async_copy(attr_h.at[pl.ds(0, k)], g0_v[b],
                                  g_sem[b]).wait()
            pltpu.make_async_copy(attr_h.at[pl.ds(0, k)], g1_v[b],
                                  g_sem[b]).wait()

        def mul(b):
            def mul_body(i2, _2):
                sl = pl.ds(i2 * 16, 16)
                a = attr_v[b][sl]
                g0_v[b][sl] = g0_v[b][sl] * a
                g1_v[b][sl] = g1_v[b][sl] * a
                return _2

            lax.fori_loop(0, k // 16, mul_body, None, unroll=4)

        def fire_scatters(b):
            pltpu.async_copy(g0_v[b], z0_sh.at[col_v[b]], sc_sem[b],
                             add=True)
            pltpu.async_copy(g1_v[b], z1_sh.at[col_v[b]], sc_sem[b],
                             add=True)

        fire_streams(0, wid * ce)

        def chunk_body(j, _):
            base = wid * ce + j * 2 * k

            @pl.when(j > 0)
            def _():
                drain_scatters(1)

            fire_streams(1, base + k)

            drain_streams(0)
            fire_gathers(0)
            mul(0)
            fire_scatters(0)

            drain_streams(1)
            fire_gathers(1)
            drain_scatters(0)

            @pl.when(j + 1 < nloop)
            def _():
                fire_streams(0, base + 2 * k)

            mul(1)
            fire_scatters(1)
            return _

        lax.fori_loop(0, nloop, chunk_body, None)
        drain_scatters(1)
        plsc.subcore_barrier()
        pltpu.sync_copy(z0_sh.at[tsl], out_h.at[c, 0, tsl])
        pltpu.sync_copy(z1_sh.at[tsl], out_h.at[c, 1, tsl])

    return prop_kernel(row, col, attr, y0, y1)


def _tc_prep(degp0, degp1, x0, x1):
    def body(d0_r, d1_r, x0_r, x1_r, dis_r, y0_r, y1_r):
        deg = d0_r[...] + d1_r[...]
        safe = jnp.where(deg > 0, deg, 1.0)
        dis = jnp.where(deg > 0, 1.0 / jnp.sqrt(safe), 0.0)
        dis_r[...] = dis
        y0_r[...] = dis * x0_r[...]
        y1_r[...] = dis * x1_r[...]

    shp = jax.ShapeDtypeStruct(degp0.shape, jnp.float32)
    return pl.pallas_call(body, out_shape=[shp, shp, shp])(degp0, degp1, x0, x1)


def _tc_mid(dis, z00, z01, z10, z11):
    def body(dis_r, z00_r, z01_r, z10_r, z11_r, t0_r, t1_r, y0_r, y1_r):
        dis = dis_r[...]
        t0 = -dis * (z00_r[...] + z10_r[...])
        t1 = -dis * (z01_r[...] + z11_r[...])
        t0_r[...] = t0
        t1_r[...] = t1
        y0_r[...] = dis * t0
        y1_r[...] = dis * t1

    shp = jax.ShapeDtypeStruct(dis.shape, jnp.float32)
    return pl.pallas_call(body, out_shape=[shp] * 4)(dis, z00, z01, z10, z11)


def _tc_final(x0, x1, xt, dis, t10, t11, z00, z01, z10, z11, params, n_valid):
    rows, lanes = x0.shape

    def body(x0_r, x1_r, xt_r, dis_r, t10_r, t11_r,
             z00_r, z01_r, z10_r, z11_r, p_r, out_r):
        dis = dis_r[...]
        tx00, tx01 = x0_r[...], x1_r[...]
        tx10, tx11 = t10_r[...], t11_r[...]
        tx20 = -2.0 * dis * (z00_r[...] + z10_r[...]) - tx00
        tx21 = -2.0 * dis * (z01_r[...] + z11_r[...]) - tx01

        def bf(v):
            return v.astype(jnp.bfloat16).astype(jnp.float32)

        def w(kk, d, c):
            return bf(p_r[kk * 4 + d * 2 + c])

        b00, b01 = bf(tx00), bf(tx01)
        b10, b11 = bf(tx10), bf(tx11)
        b20, b21 = bf(tx20), bf(tx21)
        h0 = (((b00 * w(0, 0, 0) + b01 * w(0, 1, 0))
               + (b10 * w(1, 0, 0) + b11 * w(1, 1, 0)))
              + (b20 * w(2, 0, 0) + b21 * w(2, 1, 0))) + p_r[12]
        h1 = (((b00 * w(0, 0, 1) + b01 * w(0, 1, 1))
               + (b10 * w(1, 0, 1) + b11 * w(1, 1, 1)))
              + (b20 * w(2, 0, 1) + b21 * w(2, 1, 1))) + p_r[13]
        h0 = jnp.maximum(h0, 0.0)
        h1 = jnp.maximum(h1, 0.0)
        u = (bf(xt_r[...]) * bf(p_r[14]) + bf(h0) * bf(p_r[15])
             + bf(h1) * bf(p_r[16])) + p_r[17]

        ridx = lax.broadcasted_iota(jnp.int32, (rows, lanes), 0)
        cidx = lax.broadcasted_iota(jnp.int32, (rows, lanes), 1)
        valid = (ridx * lanes + cidx) < n_valid
        um = jnp.where(valid, u, -jnp.inf)
        m = jnp.max(um)
        e = jnp.where(valid, jnp.exp(u - m), 0.0)
        out_r[...] = e / jnp.sum(e)

    vspec = pl.BlockSpec(memory_space=pltpu.VMEM)
    sspec = pl.BlockSpec(memory_space=pltpu.SMEM)
    return pl.pallas_call(
        body,
        in_specs=[vspec] * 10 + [sspec],
        out_specs=vspec,
        out_shape=jax.ShapeDtypeStruct((rows, lanes), jnp.float32),
    )(x0, x1, xt, dis, t10, t11, z00, z01, z10, z11, params)


def kernel(x, edge_index, edge_attr, cheb_w, cheb_b, W, b):
    n = x.shape[0]
    np_pad = ((n + 1023) // 1024) * 1024
    r = np_pad // 128
    row = edge_index[0]
    col = edge_index[1]

    xp = jnp.pad(x, ((0, np_pad - n), (0, 0)))
    x0 = xp[:, 0].reshape(r, 128)
    x1 = xp[:, 1].reshape(r, 128)
    xt = xp[:, 2].reshape(r, 128)

    degp = _sc_deg(row, edge_attr, np_pad)
    dis, y00, y01 = _tc_prep(degp[0].reshape(r, 128), degp[1].reshape(r, 128),
                             x0, x1)

    z1 = _sc_prop(row, col, edge_attr, y00.reshape(np_pad),
                  y01.reshape(np_pad), np_pad)
    t10, t11, y10, y11 = _tc_mid(
        dis,
        z1[0, 0].reshape(r, 128), z1[0, 1].reshape(r, 128),
        z1[1, 0].reshape(r, 128), z1[1, 1].reshape(r, 128))

    z2 = _sc_prop(row, col, edge_attr, y10.reshape(np_pad),
                  y11.reshape(np_pad), np_pad)

    params = jnp.concatenate([cheb_w.reshape(-1), cheb_b, W, b]).astype(jnp.float32)
    res = _tc_final(
        x0, x1, xt, dis, t10, t11,
        z2[0, 0].reshape(r, 128), z2[0, 1].reshape(r, 128),
        z2[1, 0].reshape(r, 128), z2[1, 1].reshape(r, 128),
        params, n)
    return res.reshape(np_pad)[:n]


# speedup vs baseline: 1.1982x; 1.1982x over previous
import functools

import jax
import jax.numpy as jnp
from jax import lax
from jax.experimental import pallas as pl
from jax.experimental.pallas import tpu as pltpu
from jax.experimental.pallas import tpu_sc as plsc

_NC = 2
_NS = 16
_NW = _NC * _NS


def _pick_chunk(per_worker: int, cap: int) -> int:
    for k in (40000, 25000, 20000, 10000, 8000, 6400, 5000, 4000, 3200, 2000,
              1600, 1000, 800, 500, 400, 250, 200, 125, 100, 64, 40, 32, 16, 8):
        if k <= cap and per_worker % k == 0 and k % 8 == 0:
            return k
    return 8


def _sc_deg(row, attr, np_pad):
    e = row.shape[0]
    ce = e // _NW
    k = _pick_chunk(ce, 40000)
    ntile = np_pad // _NS
    mesh = plsc.VectorSubcoreMesh(core_axis_name="c", subcore_axis_name="s")

    @functools.partial(
        pl.kernel,
        out_type=jax.ShapeDtypeStruct((_NC, np_pad), jnp.float32),
        mesh=mesh,
        scratch_types=[
            pltpu.VMEM_SHARED((np_pad,), jnp.float32),
            pltpu.VMEM((k,), jnp.int32),
            pltpu.VMEM((k,), jnp.float32),
            pltpu.VMEM((ntile,), jnp.float32),
        ],
    )
    def deg_kernel(row_h, attr_h, out_h, deg_sh, idx_v, val_v, zero_v):
        c = lax.axis_index("c")
        s = lax.axis_index("s")
        wid = c * _NS + s

        def zero_body(i, _):
            zero_v[pl.ds(i * 16, 16)] = jnp.zeros((16,), jnp.float32)
            return _

        lax.fori_loop(0, ntile // 16, zero_body, None, unroll=4)
        pltpu.sync_copy(zero_v, deg_sh.at[pl.ds(s * ntile, ntile)])
        plsc.subcore_barrier()

        def chunk_body(i, _):
            base = wid * ce + i * k
            pltpu.sync_copy(row_h.at[pl.ds(base, k)], idx_v)
            pltpu.sync_copy(attr_h.at[pl.ds(base, k)], val_v)
            pltpu.sync_copy(val_v, deg_sh.at[idx_v], add=True)
            return _

        lax.fori_loop(0, ce // k, chunk_body, None)
        plsc.subcore_barrier()
        pltpu.sync_copy(deg_sh.at[pl.ds(s * ntile, ntile)],
                        out_h.at[c, pl.ds(s * ntile, ntile)])

    return deg_kernel(row, attr)


def _sc_prop(row, col, attr, y0, y1, np_pad):
    e = row.shape[0]
    ce = e // _NW
    k = _pick_chunk(ce, 20000)
    ntile = np_pad // _NS
    mesh = plsc.VectorSubcoreMesh(core_axis_name="c", subcore_axis_name="s")

    @functools.partial(
        pl.kernel,
        out_type=jax.ShapeDtypeStruct((_NC, 2, np_pad), jnp.float32),
        mesh=mesh,
        scratch_types=[
            pltpu.VMEM_SHARED((np_pad,), jnp.float32),
            pltpu.VMEM_SHARED((np_pad,), jnp.float32),
            pltpu.VMEM_SHARED((np_pad,), jnp.float32),
            pltpu.VMEM_SHARED((np_pad,), jnp.float32),
            pltpu.VMEM((k,), jnp.int32),
            pltpu.VMEM((k,), jnp.int32),
            pltpu.VMEM((k,), jnp.float32),
            pltpu.VMEM((k,), jnp.float32),
            pltpu.VMEM((k,), jnp.float32),
        ],
    )
    def prop_kernel(row_h, col_h, attr_h, y0_h, y1_h, out_h,
                    y0_sh, y1_sh, z0_sh, z1_sh,
                    row_v, col_v, attr_v, g0_v, g1_v):
        c = lax.axis_index("c")
        s = lax.axis_index("s")
        wid = c * _NS + s
        tsl = pl.ds(s * ntile, ntile)

        pltpu.sync_copy(y0_h.at[tsl], y0_sh.at[tsl])
        pltpu.sync_copy(y1_h.at[tsl], y1_sh.at[tsl])

        def zero_body(i, _):
            g0_v[pl.ds(i * 16, 16)] = jnp.zeros((16,), jnp.float32)
            return _

        lax.fori_loop(0, k // 16, zero_body, None, unroll=4)
        pltpu.sync_copy(g0_v.at[pl.ds(0, ntile)], z0_sh.at[tsl])
        pltpu.sync_copy(g0_v.at[pl.ds(0, ntile)], z1_sh.at[tsl])
        plsc.subcore_barrier()

        def chunk_body(i, _):
            base = wid * ce + i * k
            pltpu.sync_copy(row_h.at[pl.ds(base, k)], row_v)
            pltpu.sync_copy(col_h.at[pl.ds(base, k)], col_v)
            pltpu.sync_copy(attr_h.at[pl.ds(base, k)], attr_v)
            pltpu.sync_copy(y0_sh.at[row_v], g0_v)
            pltpu.sync_copy(y1_sh.at[row_v], g1_v)

            def mul_body(i2, _2):
                sl = pl.ds(i2 * 16, 16)
                a = attr_v[sl]
                g0_v[sl] = g0_v[sl] * a
                g1_v[sl] = g1_v[sl] * a
                return _2

            lax.fori_loop(0, k // 16, mul_body, None, unroll=4)
            pltpu.sync_copy(g0_v, z0_sh.at[col_v], add=True)
            pltpu.sync_copy(g1_v, z1_sh.at[col_v], add=True)
            return _

        lax.fori_loop(0, ce // k, chunk_body, None)
        plsc.subcore_barrier()
        pltpu.sync_copy(z0_sh.at[tsl], out_h.at[c, 0, tsl])
        pltpu.sync_copy(z1_sh.at[tsl], out_h.at[c, 1, tsl])

    return prop_kernel(row, col, attr, y0, y1)


def _tc_prep(degp0, degp1, x0, x1):
    def body(d0_r, d1_r, x0_r, x1_r, dis_r, y0_r, y1_r):
        deg = d0_r[...] + d1_r[...]
        safe = jnp.where(deg > 0, deg, 1.0)
        dis = jnp.where(deg > 0, 1.0 / jnp.sqrt(safe), 0.0)
        dis_r[...] = dis
        y0_r[...] = dis * x0_r[...]
        y1_r[...] = dis * x1_r[...]

    shp = jax.ShapeDtypeStruct(degp0.shape, jnp.float32)
    return pl.pallas_call(body, out_shape=[shp, shp, shp])(degp0, degp1, x0, x1)


def _tc_mid(dis, z00, z01, z10, z11):
    def body(dis_r, z00_r, z01_r, z10_r, z11_r, t0_r, t1_r, y0_r, y1_r):
        dis = dis_r[...]
        t0 = -dis * (z00_r[...] + z10_r[...])
        t1 = -dis * (z01_r[...] + z11_r[...])
        t0_r[...] = t0
        t1_r[...] = t1
        y0_r[...] = dis * t0
        y1_r[...] = dis * t1

    shp = jax.ShapeDtypeStruct(dis.shape, jnp.float32)
    return pl.pallas_call(body, out_shape=[shp] * 4)(dis, z00, z01, z10, z11)


def _tc_final(x0, x1, xt, dis, t10, t11, z00, z01, z10, z11, params, n_valid):
    rows, lanes = x0.shape

    def body(x0_r, x1_r, xt_r, dis_r, t10_r, t11_r,
             z00_r, z01_r, z10_r, z11_r, p_r, out_r):
        dis = dis_r[...]
        tx00, tx01 = x0_r[...], x1_r[...]
        tx10, tx11 = t10_r[...], t11_r[...]
        tx20 = -2.0 * dis * (z00_r[...] + z10_r[...]) - tx00
        tx21 = -2.0 * dis * (z01_r[...] + z11_r[...]) - tx01

        def bf(v):
            return v.astype(jnp.bfloat16).astype(jnp.float32)

        def w(kk, d, c):
            return bf(p_r[kk * 4 + d * 2 + c])

        b00, b01 = bf(tx00), bf(tx01)
        b10, b11 = bf(tx10), bf(tx11)
        b20, b21 = bf(tx20), bf(tx21)
        h0 = (((b00 * w(0, 0, 0) + b01 * w(0, 1, 0))
               + (b10 * w(1, 0, 0) + b11 * w(1, 1, 0)))
              + (b20 * w(2, 0, 0) + b21 * w(2, 1, 0))) + p_r[12]
        h1 = (((b00 * w(0, 0, 1) + b01 * w(0, 1, 1))
               + (b10 * w(1, 0, 1) + b11 * w(1, 1, 1)))
              + (b20 * w(2, 0, 1) + b21 * w(2, 1, 1))) + p_r[13]
        h0 = jnp.maximum(h0, 0.0)
        h1 = jnp.maximum(h1, 0.0)
        u = (bf(xt_r[...]) * bf(p_r[14]) + bf(h0) * bf(p_r[15])
             + bf(h1) * bf(p_r[16])) + p_r[17]

        ridx = lax.broadcasted_iota(jnp.int32, (rows, lanes), 0)
        cidx = lax.broadcasted_iota(jnp.int32, (rows, lanes), 1)
        valid = (ridx * lanes + cidx) < n_valid
        um = jnp.where(valid, u, -jnp.inf)
        m = jnp.max(um)
        e = jnp.where(valid, jnp.exp(u - m), 0.0)
        out_r[...] = e / jnp.sum(e)

    vspec = pl.BlockSpec(memory_space=pltpu.VMEM)
    sspec = pl.BlockSpec(memory_space=pltpu.SMEM)
    return pl.pallas_call(
        body,
        in_specs=[vspec] * 10 + [sspec],
        out_specs=vspec,
        out_shape=jax.ShapeDtypeStruct((rows, lanes), jnp.float32),
    )(x0, x1, xt, dis, t10, t11, z00, z01, z10, z11, params)


def kernel(x, edge_index, edge_attr, cheb_w, cheb_b, W, b):
    n = x.shape[0]
    np_pad = ((n + 1023) // 1024) * 1024
    r = np_pad // 128
    row = edge_index[0]
    col = edge_index[1]

    xp = jnp.pad(x, ((0, np_pad - n), (0, 0)))
    x0 = xp[:, 0].reshape(r, 128)
    x1 = xp[:, 1].reshape(r, 128)
    xt = xp[:, 2].reshape(r, 128)

    degp = _sc_deg(row, edge_attr, np_pad)
    dis, y00, y01 = _tc_prep(degp[0].reshape(r, 128), degp[1].reshape(r, 128),
                             x0, x1)

    z1 = _sc_prop(row, col, edge_attr, y00.reshape(np_pad),
                  y01.reshape(np_pad), np_pad)
    t10, t11, y10, y11 = _tc_mid(
        dis,
        z1[0, 0].reshape(r, 128), z1[0, 1].reshape(r, 128),
        z1[1, 0].reshape(r, 128), z1[1, 1].reshape(r, 128))

    z2 = _sc_prop(row, col, edge_attr, y10.reshape(np_pad),
                  y11.reshape(np_pad), np_pad)

    params = jnp.concatenate([cheb_w.reshape(-1), cheb_b, W, b]).astype(jnp.float32)
    res = _tc_final(
        x0, x1, xt, dis, t10, t11,
        z2[0, 0].reshape(r, 128), z2[0, 1].reshape(r, 128),
        z2[1, 0].reshape(r, 128), z2[1, 1].reshape(r, 128),
        params, n)
    return res.reshape(np_pad)[:n]
